# baseline (speedup 1.0000x reference)
import jax
import jax.numpy as jnp
from jax.experimental import pallas as pl
from jax.experimental.pallas import tpu as pltpu

_NUM_PID = 15080
_SCALAR = 10.0
_ROWS = 64
_CHUNK = 1664


def _oim_kernel(ids_ref, feat_ref, lut_ref, out_ref, s_ref, p_ref):
    j = pl.program_id(0)
    nc = pl.num_programs(0)

    @pl.when(j == 0)
    def _init():
        s_ref[...] = jnp.zeros_like(s_ref)
        p_ref[...] = jnp.zeros_like(p_ref)

    feat = feat_ref[...]
    lut_c = lut_ref[...]
    logits = _SCALAR * jax.lax.dot_general(
        feat, lut_c, (((1,), (1,)), ((), ())),
        preferred_element_type=jnp.float32)

    base = j * _CHUNK
    col = base + jax.lax.broadcasted_iota(jnp.int32, (_ROWS, _CHUNK), 1)
    valid = col < _NUM_PID

    e = jnp.where(valid, jnp.exp(logits - _SCALAR), 0.0)
    s_ref[...] += e.reshape(_ROWS, _CHUNK // 128, 128).sum(axis=1)

    pids = ids_ref[:, :1]
    row_ok = pids > -1
    safe = jnp.where(row_ok, pids, 0)
    pick = jnp.where(col == safe, logits, 0.0)
    p_ref[...] += pick.reshape(_ROWS, _CHUNK // 128, 128).sum(axis=1)

    @pl.when(j == nc - 1)
    def _fin():
        s_tot = s_ref[...].sum(axis=1, keepdims=True)
        p_tot = p_ref[...].sum(axis=1, keepdims=True)
        lse = jnp.log(s_tot) + _SCALAR
        per_row = jnp.where(row_ok, lse - p_tot, 0.0)
        cnt = jnp.sum(row_ok.astype(jnp.float32))
        out_ref[0, 0] = jnp.sum(per_row) / cnt


def kernel(features, gt_labels, lut):
    pids = gt_labels.reshape(-1, gt_labels.shape[-1])[:, -1].astype(jnp.int32)
    ids2d = jnp.broadcast_to(pids[:, None], (_ROWS, 128))
    nc = pl.cdiv(_NUM_PID, _CHUNK)
    loss = pl.pallas_call(
        _oim_kernel,
        grid=(nc,),
        in_specs=[
            pl.BlockSpec((_ROWS, 128), lambda j: (0, 0)),
            pl.BlockSpec((_ROWS, features.shape[1]), lambda j: (0, 0)),
            pl.BlockSpec((_CHUNK, lut.shape[1]), lambda j: (j, 0)),
        ],
        out_specs=pl.BlockSpec(memory_space=pltpu.SMEM),
        out_shape=jax.ShapeDtypeStruct((1, 1), jnp.float32),
        scratch_shapes=[
            pltpu.VMEM((_ROWS, 128), jnp.float32),
            pltpu.VMEM((_ROWS, 128), jnp.float32),
        ],
    )(ids2d, features, lut)
    return loss[0, 0]


# speedup vs baseline: 1.0531x; 1.0531x over previous
import jax
import jax.numpy as jnp
from jax.experimental import pallas as pl
from jax.experimental.pallas import tpu as pltpu

_NUM_PID = 15080
_SCALAR = 10.0
_ROWS = 64
_CHUNK = 1280


def _oim_kernel(ids_ref, feat_ref, lut_ref, out_ref, s_ref, p_ref):
    j = pl.program_id(0)
    nc = pl.num_programs(0)

    @pl.when(j == 0)
    def _init():
        s_ref[...] = jnp.zeros_like(s_ref)
        p_ref[...] = jnp.zeros_like(p_ref)

    feat = feat_ref[...]
    lut_c = lut_ref[...]
    logits = _SCALAR * jax.lax.dot_general(
        feat, lut_c, (((1,), (1,)), ((), ())),
        preferred_element_type=jnp.float32)

    base = j * _CHUNK
    col = base + jax.lax.broadcasted_iota(jnp.int32, (_ROWS, _CHUNK), 1)
    valid = col < _NUM_PID

    e = jnp.where(valid, jnp.exp(logits - _SCALAR), 0.0)
    s_ref[...] += e.reshape(_ROWS, _CHUNK // 128, 128).sum(axis=1)

    pids = ids_ref[:, :1]
    row_ok = pids > -1
    safe = jnp.where(row_ok, pids, 0)
    pick = jnp.where(col == safe, logits, 0.0)
    p_ref[...] += pick.reshape(_ROWS, _CHUNK // 128, 128).sum(axis=1)

    @pl.when(j == nc - 1)
    def _fin():
        s_tot = s_ref[...].sum(axis=1, keepdims=True)
        p_tot = p_ref[...].sum(axis=1, keepdims=True)
        lse = jnp.log(s_tot) + _SCALAR
        per_row = jnp.where(row_ok, lse - p_tot, 0.0)
        cnt = jnp.sum(row_ok.astype(jnp.float32))
        out_ref[0, 0] = jnp.sum(per_row) / cnt


def kernel(features, gt_labels, lut):
    pids = gt_labels.reshape(-1, gt_labels.shape[-1])[:, -1].astype(jnp.int32)
    ids2d = jnp.broadcast_to(pids[:, None], (_ROWS, 128))
    nc = pl.cdiv(_NUM_PID, _CHUNK)
    loss = pl.pallas_call(
        _oim_kernel,
        grid=(nc,),
        in_specs=[
            pl.BlockSpec((_ROWS, 128), lambda j: (0, 0)),
            pl.BlockSpec((_ROWS, features.shape[1]), lambda j: (0, 0)),
            pl.BlockSpec((_CHUNK, lut.shape[1]), lambda j: (j, 0)),
        ],
        out_specs=pl.BlockSpec(memory_space=pltpu.SMEM),
        out_shape=jax.ShapeDtypeStruct((1, 1), jnp.float32),
        scratch_shapes=[
            pltpu.VMEM((_ROWS, 128), jnp.float32),
            pltpu.VMEM((_ROWS, 128), jnp.float32),
        ],
    )(ids2d, features, lut)
    return loss[0, 0]


# speedup vs baseline: 1.0568x; 1.0035x over previous
import jax
import jax.numpy as jnp
from jax.experimental import pallas as pl
from jax.experimental.pallas import tpu as pltpu

_NUM_PID = 15080
_SCALAR = 10.0
_ROWS = 64
_CHUNK = 1536


def _oim_kernel(ids_ref, feat_ref, lut_ref, out_ref, s_ref, p_ref):
    j = pl.program_id(0)
    nc = pl.num_programs(0)

    @pl.when(j == 0)
    def _init():
        s_ref[...] = jnp.zeros_like(s_ref)
        p_ref[...] = jnp.zeros_like(p_ref)

    feat = feat_ref[...]
    lut_c = lut_ref[...]
    logits = _SCALAR * jax.lax.dot_general(
        feat, lut_c, (((1,), (1,)), ((), ())),
        preferred_element_type=jnp.float32)

    base = j * _CHUNK
    col = base + jax.lax.broadcasted_iota(jnp.int32, (_ROWS, _CHUNK), 1)
    valid = col < _NUM_PID

    e = jnp.where(valid, jnp.exp(logits - _SCALAR), 0.0)
    s_ref[...] += e.reshape(_ROWS, _CHUNK // 128, 128).sum(axis=1)

    pids = ids_ref[:, :1]
    row_ok = pids > -1
    safe = jnp.where(row_ok, pids, 0)
    pick = jnp.where(col == safe, logits, 0.0)
    p_ref[...] += pick.reshape(_ROWS, _CHUNK // 128, 128).sum(axis=1)

    @pl.when(j == nc - 1)
    def _fin():
        s_tot = s_ref[...].sum(axis=1, keepdims=True)
        p_tot = p_ref[...].sum(axis=1, keepdims=True)
        lse = jnp.log(s_tot) + _SCALAR
        per_row = jnp.where(row_ok, lse - p_tot, 0.0)
        cnt = jnp.sum(row_ok.astype(jnp.float32))
        out_ref[0, 0] = jnp.sum(per_row) / cnt


def kernel(features, gt_labels, lut):
    pids = gt_labels.reshape(-1, gt_labels.shape[-1])[:, -1].astype(jnp.int32)
    ids2d = jnp.broadcast_to(pids[:, None], (_ROWS, 128))
    nc = pl.cdiv(_NUM_PID, _CHUNK)
    loss = pl.pallas_call(
        _oim_kernel,
        grid=(nc,),
        in_specs=[
            pl.BlockSpec((_ROWS, 128), lambda j: (0, 0)),
            pl.BlockSpec((_ROWS, features.shape[1]), lambda j: (0, 0)),
            pl.BlockSpec((_CHUNK, lut.shape[1]), lambda j: (j, 0)),
        ],
        out_specs=pl.BlockSpec(memory_space=pltpu.SMEM),
        out_shape=jax.ShapeDtypeStruct((1, 1), jnp.float32),
        scratch_shapes=[
            pltpu.VMEM((_ROWS, 128), jnp.float32),
            pltpu.VMEM((_ROWS, 128), jnp.float32),
        ],
    )(ids2d, features, lut)
    return loss[0, 0]
